# baseline (speedup 1.0000x reference)
import functools

import numpy as np
import jax
import jax.numpy as jnp
from jax.experimental import pallas as pl
from jax.experimental.pallas import tpu as pltpu

N_BINS_K = 15


def _ece_kernel(logits_ref, labels_ref, out_ref, acc_ref, *, n_total, n_blocks):
    i = pl.program_id(0)

    @pl.when(i == 0)
    def _init():
        acc_ref[...] = jnp.zeros_like(acc_ref)

    x = logits_ref[...]
    r, c = x.shape
    m = jnp.max(x, axis=1, keepdims=True)
    s = jnp.sum(jnp.exp(x - m), axis=1, keepdims=True)
    conf = 1.0 / s

    iota = jax.lax.broadcasted_iota(jnp.int32, (r, c), 1)
    pred = jnp.min(jnp.where(x == m, iota, c), axis=1, keepdims=True)
    is_correct = (pred == labels_ref[...]).astype(jnp.float32)

    biota = jax.lax.broadcasted_iota(jnp.int32, (1, N_BINS_K), 1).astype(jnp.float32)
    lowers = biota / np.float32(N_BINS_K)
    uppers = (biota + 1.0) / np.float32(N_BINS_K)
    mask = ((conf > lowers) & (conf <= uppers)).astype(jnp.float32)

    cnt = jnp.sum(mask, axis=0, keepdims=True)
    csum = jnp.sum(conf * mask, axis=0, keepdims=True)
    asum = jnp.sum(is_correct * mask, axis=0, keepdims=True)
    acc_ref[...] += jnp.concatenate([cnt, csum, asum], axis=0)

    @pl.when(i == n_blocks - 1)
    def _finish():
        tot = acc_ref[0:1, :]
        cs = acc_ref[1:2, :]
        asm = acc_ref[2:3, :]
        denom = jnp.maximum(tot, 1.0)
        gap = jnp.abs(cs / denom - asm / denom)
        contrib = jnp.where(tot > 0, gap * (tot / np.float32(n_total)), 0.0)
        out_ref[...] = jnp.sum(contrib, axis=(0, 1), keepdims=True)


def kernel(logits, labels):
    n, c = logits.shape
    block = 1
    for cand in (8000, 8192, 4096, 4000, 2048, 2000, 1024, 1000, 512, 500,
                 256, 250, 128, 125, 100, 64, 50, 32, 25, 16, 10, 8, 5, 4, 2):
        if n % cand == 0:
            block = cand
            break
    n_blocks = n // block
    labels2d = labels.astype(jnp.int32).reshape(n, 1)

    out = pl.pallas_call(
        functools.partial(_ece_kernel, n_total=n, n_blocks=n_blocks),
        grid=(n_blocks,),
        in_specs=[
            pl.BlockSpec((block, c), lambda i: (i, 0)),
            pl.BlockSpec((block, 1), lambda i: (i, 0)),
        ],
        out_specs=pl.BlockSpec((1, 1), lambda i: (0, 0)),
        out_shape=jax.ShapeDtypeStruct((1, 1), jnp.float32),
        scratch_shapes=[pltpu.VMEM((3, N_BINS_K), jnp.float32)],
    )(logits, labels2d)
    return out.reshape(1)
